# baseline (speedup 1.0000x reference)
import functools

import jax
import jax.numpy as jnp
from jax import lax
from jax.experimental import pallas as pl
from jax.experimental.pallas import tpu as pltpu
from jax.experimental.pallas import tpu_sc as plsc

_DOMS = (100, 100, 100)
_B = 16384

_NC = 2
_NS = 16
_NW = _NC * _NS
_BPW = _B // _NW
_CHUNK = 128
_NCHUNK = _BPW // _CHUNK


def _sc_body(x_hbm, w_hbm, out_hbm, xv, idxv, rows, sem):
    wid = lax.axis_index("s") * _NC + lax.axis_index("c")
    base = wid * _BPW

    for d in range(3):
        pltpu.sync_copy(
            x_hbm.at[pl.ds(d * _B + base, _BPW)], xv.at[pl.ds(d * _BPW, _BPW)]
        )

    for g in range(_BPW // 16):
        i0 = xv[pl.ds(g * 16, 16)]
        i1 = xv[pl.ds(_BPW + g * 16, 16)]
        i2 = xv[pl.ds(2 * _BPW + g * 16, 16)]
        flat = i0 * jnp.int32(_DOMS[1] * _DOMS[2]) + i1 * jnp.int32(_DOMS[2]) + i2
        j, k = divmod(g, _CHUNK // 16)
        idxv[j, pl.ds(k * 16, 16)] = flat

    copies = [
        pltpu.async_copy(w_hbm.at[idxv.at[j]], rows.at[pl.ds(j * _CHUNK, _CHUNK)], sem)
        for j in range(_NCHUNK)
    ]
    for c in copies:
        c.wait()

    pltpu.sync_copy(rows, out_hbm.at[pl.ds(base, _BPW)])


@functools.partial(jax.jit)
def _sc_gather(x_flat, w_flat):
    mesh = plsc.VectorSubcoreMesh(core_axis_name="c", subcore_axis_name="s")
    return pl.kernel(
        _sc_body,
        mesh=mesh,
        out_type=jax.ShapeDtypeStruct((_B,), jnp.float32),
        scratch_types=[
            pltpu.VMEM((3 * _BPW,), jnp.int32),
            pltpu.VMEM((_NCHUNK, _CHUNK), jnp.int32),
            pltpu.VMEM((_BPW,), jnp.float32),
            pltpu.SemaphoreType.DMA,
        ],
    )(x_flat, w_flat)


def kernel(x, W):
    x_t = x.astype(jnp.int32).T.reshape(-1)
    w_flat = W.reshape(-1)
    return _sc_gather(x_t, w_flat).reshape(_B, 1)
